# baseline (speedup 1.0000x reference)
import jax
import jax.numpy as jnp
from jax.experimental import pallas as pl

TILES = 6
_BF = jnp.bfloat16


def _nsum(x):
    H, W, F = x.shape
    zr = jnp.zeros((1, W, F), x.dtype)
    zc = jnp.zeros((H, 1, F), x.dtype)
    return (
        jnp.concatenate([x[1:], zr], axis=0)
        + jnp.concatenate([zr, x[:-1]], axis=0)
        + jnp.concatenate([x[:, 1:], zc], axis=1)
        + jnp.concatenate([zc, x[:, :-1]], axis=1)
    )


def _sage_pre(x, rdeg, Wsn, b, odt=_BF):
    H, W, F = x.shape
    a = _nsum(x) * rdeg[:, :, None]
    cat = jnp.concatenate([x, a], axis=-1).reshape(H * W, 2 * F)
    o = jnp.dot(cat, Wsn, preferred_element_type=jnp.float32)
    o = jnp.maximum(o + b, 0.0).astype(odt)
    return o.reshape(H, W, -1)


def _sage_post_skip(before, lo, rdeg, cat1, upWc, ball, odt=_BF):
    H, W, F1 = before.shape
    h, w, Fl = lo.shape
    Fo = cat1.shape[1] // 2
    y0 = jnp.dot(before.reshape(H * W, F1), cat1,
                 preferred_element_type=jnp.float32).reshape(H, W, 2 * Fo)
    q = jnp.dot(lo.reshape(h * w, Fl), upWc,
                preferred_element_type=jnp.float32).reshape(h, w, 2 * Fo)
    y = (y0 + _rep2(q)).astype(_BF)
    ys = y[:, :, :Fo]
    yn = y[:, :, Fo:]
    o = ys + _nsum(yn) * rdeg[:, :, None] + ball
    return jnp.maximum(o, 0.0).astype(odt)


def _pool(x):
    H, W, F = x.shape
    x2 = x.reshape(H // 2, 2, W, F)
    a = x2[:, 0] + x2[:, 1]
    m = a.reshape(H // 2, W // 2, 2 * F)
    return (m[:, :, :F] + m[:, :, F:]) * jnp.asarray(0.25, x.dtype)


def _rep2(y):
    h, w, F = y.shape
    z = jnp.concatenate([y, y], axis=-1).reshape(h, 2 * w, F)
    z = jnp.broadcast_to(z[:, None], (h, 2, 2 * w, F)).reshape(2 * h, 2 * w, F)
    return z


def _unet_body(x_ref, rd0_ref, rd1_ref, rd2_ref, *rest):
    (
        w10, c10, w11, c11,
        w20, c20, w21, c21,
        wl0, cl0, wl1, cl1,
        k1a, k1u, d20,
        w2b, d21,
        k0a, k0u, d10,
        w1b, d11,
        out_ref,
    ) = rest
    x0 = x_ref[0]
    rd0, rd1, rd2 = rd0_ref[0], rd1_ref[0], rd2_ref[0]

    b0 = _sage_pre(_sage_pre(x0, rd0, w10[...], c10[...]),
                   rd0, w11[...], c11[...])
    b1 = _sage_pre(_sage_pre(_pool(b0), rd1, w20[...], c20[...]),
                   rd1, w21[...], c21[...])
    lo = _sage_pre(_sage_pre(_pool(b1), rd2, wl0[...], cl0[...]),
                   rd2, wl1[...], cl1[...])

    o1 = _sage_post_skip(b1, lo, rd1, k1a[...], k1u[...], d20[...])
    o1 = _sage_pre(o1, rd1, w2b[...], d21[...])

    o0 = _sage_post_skip(b0, o1, rd0, k0a[...], k0u[...], d10[...])
    out_ref[0] = _sage_pre(o0, rd0, w1b[...], d11[...], odt=jnp.float32)


def kernel(inputs, params, graphs):
    B, T, H, W, F = inputs.shape
    x = inputs.reshape(T, H, W, F).astype(_BF)
    rdegs = []
    nx = H
    for g in graphs:
        rdegs.append((1.0 / g[2]).reshape(T, nx, nx).astype(_BF))
        nx //= 2

    p0, p1 = params["level0"], params["level1"]

    def pre(t):
        Ws, Wn, b = t
        return [jnp.concatenate([Ws, Wn], axis=0).astype(_BF),
                b.reshape(1, -1)]

    def post_skip(t, upW, upb, f1):
        Ws, Wn, b = t
        cat = jnp.concatenate([Ws, Wn], axis=1)
        cat1, cat2 = cat[:f1], cat[f1:]
        Fo = Ws.shape[1]
        upc = jnp.dot(upb.reshape(1, -1), cat2)
        ball = (b.reshape(1, -1) + upc[:, :Fo] + upc[:, Fo:]).astype(_BF)
        return [cat1.astype(_BF),
                jnp.dot(upW, cat2).astype(_BF),
                ball]

    weights = (
        pre(p0["conv1"]["c0"]) + pre(p0["conv1"]["c1"])
        + pre(p1["conv1"]["c0"]) + pre(p1["conv1"]["c1"])
        + pre(p1["lower"]["c0"]) + pre(p1["lower"]["c1"])
        + post_skip(p1["conv2"]["c0"], p1["upW"], p1["upb"], 256)
        + pre(p1["conv2"]["c1"])
        + post_skip(p0["conv2"]["c0"], p0["upW"], p0["upb"], 128)
        + pre(p0["conv2"]["c1"])
    )

    def tile_spec(a):
        s = a.shape
        return pl.BlockSpec((1,) + s[1:], lambda t: (t,) + (0,) * (len(s) - 1))

    def full_spec(a):
        nd = a.ndim
        return pl.BlockSpec(a.shape, lambda t, _n=nd: (0,) * _n)

    Fo = p0["conv2"]["c1"][0].shape[-1]
    out = pl.pallas_call(
        _unet_body,
        grid=(T,),
        in_specs=[tile_spec(x)] + [tile_spec(r) for r in rdegs]
        + [full_spec(wa) for wa in weights],
        out_specs=pl.BlockSpec((1, H, W, Fo), lambda t: (t, 0, 0, 0)),
        out_shape=jax.ShapeDtypeStruct((T, H, W, Fo), jnp.float32),
    )(x, *rdegs, *weights)
    return out[None]


# speedup vs baseline: 1.0470x; 1.0470x over previous
import jax
import jax.numpy as jnp
from jax.experimental import pallas as pl

TILES = 6
_BF = jnp.float32


def _nsum(x):
    H, W, F = x.shape
    zr = jnp.zeros((1, W, F), x.dtype)
    zc = jnp.zeros((H, 1, F), x.dtype)
    return (
        jnp.concatenate([x[1:], zr], axis=0)
        + jnp.concatenate([zr, x[:-1]], axis=0)
        + jnp.concatenate([x[:, 1:], zc], axis=1)
        + jnp.concatenate([zc, x[:, :-1]], axis=1)
    )


def _sage_pre(x, rdeg, Wsn, b):
    H, W, F = x.shape
    a = _nsum(x) * rdeg[:, :, None]
    cat = jnp.concatenate([x, a], axis=-1).reshape(H * W, 2 * F)
    o = jnp.dot(cat.astype(_BF), Wsn, preferred_element_type=jnp.float32)
    o = jnp.maximum(o + b, 0.0)
    return o.reshape(H, W, -1)


def _sage_post_skip(before, lo, rdeg, cat1, upWc, ball):
    H, W, F1 = before.shape
    h, w, Fl = lo.shape
    Fo = cat1.shape[1] // 2
    y0 = jnp.dot(before.reshape(H * W, F1).astype(_BF), cat1,
                 preferred_element_type=jnp.float32).reshape(H, W, 2 * Fo)
    q = jnp.dot(lo.reshape(h * w, Fl).astype(_BF), upWc,
                preferred_element_type=jnp.float32).reshape(h, w, 2 * Fo)
    y0s, y0n = y0[:, :, :Fo], y0[:, :, Fo:]
    qs, qn = q[:, :, :Fo], q[:, :, Fo:]
    zr = jnp.zeros((1, w, Fo), q.dtype)
    zc = jnp.zeros((h, 1, Fo), q.dtype)
    qv0 = qn + jnp.concatenate([zr, qn[:-1]], axis=0)
    qv1 = qn + jnp.concatenate([qn[1:], zr], axis=0)
    qh0 = qn + jnp.concatenate([zc, qn[:, :-1]], axis=1)
    qh1 = qn + jnp.concatenate([qn[:, 1:], zc], axis=1)
    sv = jnp.concatenate([qv0[:, None], qv1[:, None]],
                         axis=1).reshape(2 * h, w, Fo)
    sv = jnp.concatenate([sv, sv], axis=-1).reshape(2 * h, 2 * w, Fo)
    sh = jnp.concatenate([qh0, qh1], axis=-1).reshape(h, 2 * w, Fo)
    sh = jnp.broadcast_to(sh[:, None], (h, 2, 2 * w, Fo)
                          ).reshape(2 * h, 2 * w, Fo)
    o = y0s + _rep2(qs) + (_nsum(y0n) + sv + sh) * rdeg[:, :, None] + ball
    return jnp.maximum(o, 0.0)


def _pool(x):
    H, W, F = x.shape
    x2 = x.reshape(H // 2, 2, W, F)
    a = x2[:, 0] + x2[:, 1]
    m = a.reshape(H // 2, W // 2, 2 * F)
    return m[:, :, :F] + m[:, :, F:]


def _rep2(y):
    h, w, F = y.shape
    z = jnp.concatenate([y, y], axis=-1).reshape(h, 2 * w, F)
    z = jnp.broadcast_to(z[:, None], (h, 2, 2 * w, F)).reshape(2 * h, 2 * w, F)
    return z


def _unet_body(x_ref, rd0_ref, rd1_ref, rd2_ref, *rest):
    (
        w10, c10, w11, c11,
        w20, c20, w21, c21,
        wl0, cl0, wl1, cl1,
        k1a, k1u, k1b,
        w2b, d21,
        k0a, k0u, k0b,
        w1b, d11,
        out_ref,
    ) = rest
    x0 = x_ref[0]
    rd0, rd1, rd2 = rd0_ref[0], rd1_ref[0], rd2_ref[0]

    b0 = _sage_pre(_sage_pre(x0, rd0, w10[...], c10[...]),
                   rd0, w11[...], c11[...])
    b1 = _sage_pre(_sage_pre(_pool(b0), rd1, w20[...], c20[...]),
                   rd1, w21[...], c21[...])
    lo = _sage_pre(_sage_pre(_pool(b1), rd2, wl0[...], cl0[...]),
                   rd2, wl1[...], cl1[...])

    o1 = _sage_post_skip(b1, lo, rd1, k1a[...], k1u[...], k1b[...])
    o1 = _sage_pre(o1, rd1, w2b[...], d21[...])

    o0 = _sage_post_skip(b0, o1, rd0, k0a[...], k0u[...], k0b[...])
    out_ref[0] = _sage_pre(o0, rd0, w1b[...], d11[...])


def kernel(inputs, params, graphs):
    B, T, H, W, F = inputs.shape
    x = inputs.reshape(T, H, W, F)
    rdegs = []
    nx = H
    for g in graphs:
        rdegs.append((1.0 / g[2]).reshape(T, nx, nx))
        nx //= 2

    p0, p1 = params["level0"], params["level1"]

    def pre(t, scale=1.0):
        Ws, Wn, b = t
        return [(jnp.concatenate([Ws, Wn], axis=0) * scale).astype(_BF),
                b.reshape(1, -1)]

    def post_skip(t, upW, upb, f1):
        Ws, Wn, b = t
        cat = jnp.concatenate([Ws, Wn], axis=1)
        cat1, cat2 = cat[:f1], cat[f1:]
        Fo = Ws.shape[1]
        upc = jnp.dot(upb.reshape(1, -1), cat2)
        ball = b.reshape(1, -1) + upc[:, :Fo] + upc[:, Fo:]
        return [cat1.astype(_BF),
                jnp.dot(upW, cat2).astype(_BF),
                ball]

    weights = (
        pre(p0["conv1"]["c0"]) + pre(p0["conv1"]["c1"])
        + pre(p1["conv1"]["c0"], scale=0.25) + pre(p1["conv1"]["c1"])
        + pre(p1["lower"]["c0"], scale=0.25) + pre(p1["lower"]["c1"])
        + post_skip(p1["conv2"]["c0"], p1["upW"], p1["upb"], 256)
        + pre(p1["conv2"]["c1"])
        + post_skip(p0["conv2"]["c0"], p0["upW"], p0["upb"], 128)
        + pre(p0["conv2"]["c1"])
    )

    def tile_spec(a):
        s = a.shape
        return pl.BlockSpec((1,) + s[1:], lambda t: (t,) + (0,) * (len(s) - 1))

    def full_spec(a):
        nd = a.ndim
        return pl.BlockSpec(a.shape, lambda t, _n=nd: (0,) * _n)

    Fo = p0["conv2"]["c1"][0].shape[-1]
    out = pl.pallas_call(
        _unet_body,
        grid=(T,),
        in_specs=[tile_spec(x)] + [tile_spec(r) for r in rdegs]
        + [full_spec(wa) for wa in weights],
        out_specs=pl.BlockSpec((1, H, W, Fo), lambda t: (t, 0, 0, 0)),
        out_shape=jax.ShapeDtypeStruct((T, H, W, Fo), jnp.float32),
    )(x, *rdegs, *weights)
    return out[None]
